# Initial kernel scaffold; baseline (speedup 1.0000x reference)
import jax
import jax.numpy as jnp
from jax.experimental import pallas as pl


def kernel(node_emb, weight, bias, scale, batch):
    raise NotImplementedError("write your pallas kernel here")


# speedup vs baseline: 13.6913x; 13.6913x over previous
import functools

import jax
import jax.numpy as jnp
from jax import lax
from jax.experimental import pallas as pl
from jax.experimental.pallas import tpu as pltpu

G = 256
BLK = 2000


def _stats_body(x_ref, batch_ref, w_ref, b_ref, s_ref, rb_ref, acc_ref, cnt_ref):
    i = pl.program_id(0)
    nb = pl.num_programs(0)

    @pl.when(i == 0)
    def _init():
        acc_ref[...] = jnp.zeros_like(acc_ref)
        cnt_ref[...] = jnp.zeros_like(cnt_ref)

    bb = batch_ref[0, 0, :]
    onehot = (bb[:, None] == lax.broadcasted_iota(jnp.int32, (BLK, G), 1)
              ).astype(jnp.float32)
    x = x_ref[...]
    xcat = jnp.concatenate([x, x * x], axis=1)
    acc_ref[...] += lax.dot_general(
        onehot, xcat, (((0,), (0,)), ((), ())),
        preferred_element_type=jnp.float32)
    cnt_ref[...] += jnp.sum(onehot, axis=0)[None, :]

    @pl.when(i == nb - 1)
    def _finalize():
        d = acc_ref.shape[1] // 2
        s1 = acc_ref[:, :d]
        s2 = acc_ref[:, d:]
        cnt = cnt_ref[0, :][:, None]
        denom = jnp.maximum(cnt, 1.0)
        a = (s1 / denom) * s_ref[...]
        var = (s2 - 2.0 * a * s1 + cnt * a * a) / denom
        r = w_ref[...] * lax.rsqrt(var + 1e-8)
        bp = b_ref[...] - a * r
        rb_ref[...] = jnp.concatenate([r, bp], axis=1)


def _norm_body(x_ref, batch_ref, rb_ref, out_ref):
    bb = batch_ref[0, 0, :]
    onehot = (bb[:, None] == lax.broadcasted_iota(jnp.int32, (BLK, G), 1)
              ).astype(jnp.float32)
    g = lax.dot_general(onehot, rb_ref[...], (((1,), (0,)), ((), ())),
                        preferred_element_type=jnp.float32)
    x = x_ref[...]
    d = x.shape[1]
    out_ref[...] = x * g[:, :d] + g[:, d:]


@jax.jit
def kernel(node_emb, weight, bias, scale, batch):
    n, d = node_emb.shape
    nb = n // BLK
    batch3 = batch.astype(jnp.int32).reshape(nb, 1, BLK)
    w2 = weight.reshape(1, d)
    b2 = bias.reshape(1, d)
    s2 = scale.reshape(1, d)

    rb = pl.pallas_call(
        _stats_body,
        grid=(nb,),
        in_specs=[
            pl.BlockSpec((BLK, d), lambda i: (i, 0)),
            pl.BlockSpec((1, 1, BLK), lambda i: (i, 0, 0)),
            pl.BlockSpec((1, d), lambda i: (0, 0)),
            pl.BlockSpec((1, d), lambda i: (0, 0)),
            pl.BlockSpec((1, d), lambda i: (0, 0)),
        ],
        out_specs=pl.BlockSpec((G, 2 * d), lambda i: (0, 0)),
        out_shape=jax.ShapeDtypeStruct((G, 2 * d), jnp.float32),
        scratch_shapes=[
            pltpu.VMEM((G, 2 * d), jnp.float32),
            pltpu.VMEM((1, G), jnp.float32),
        ],
    )(node_emb, batch3, w2, b2, s2)

    out = pl.pallas_call(
        _norm_body,
        grid=(nb,),
        in_specs=[
            pl.BlockSpec((BLK, d), lambda i: (i, 0)),
            pl.BlockSpec((1, 1, BLK), lambda i: (i, 0, 0)),
            pl.BlockSpec((G, 2 * d), lambda i: (0, 0)),
        ],
        out_specs=pl.BlockSpec((BLK, d), lambda i: (i, 0)),
        out_shape=jax.ShapeDtypeStruct((n, d), jnp.float32),
    )(node_emb, batch3, rb)
    return out
